# baseline (speedup 1.0000x reference)
import functools

import jax
import jax.numpy as jnp
from jax import lax
from jax.experimental import pallas as pl
from jax.experimental.pallas import tpu as pltpu
from jax.experimental.pallas import tpu_sc as plsc

N = 10000
D = 128
H = 8
F = 16
HF = H * F
E = 320000
ET = E + N

NEG_SLOPE = 0.2

NC = 2
NS = 16
K = 16
BLOCKS = -(-ET // (NS * K))
BLOCKS += BLOCKS % 2
ETP = NS * K * BLOCKS
NBLK = NS * BLOCKS
HALF = 5000
NRH = 5120
RPT = NRH // NS
JUNK = NRH - 1
DR = 384
DPT = DR // NS

RB = 1000
GRID = N // RB


def _node0_body(x_ref, wl_ref, wr_ref, xl_ref, xr_ref):
    x = x_ref[...]
    xl_ref[...] = jnp.dot(x, wl_ref[...], preferred_element_type=jnp.float32)
    xr_ref[...] = jnp.dot(x, wr_ref[...], preferred_element_type=jnp.float32)


def _combine(pm_ref, pd_ref, xp_ref, b_ref, p_ref, relu):
    msg = pm_ref[0, :, :]
    den = pd_ref[0, :, :]
    recip = 1.0 / (den + 1e-16)
    rb = jnp.dot(recip, p_ref[...], preferred_element_type=jnp.float32)
    y = msg * rb + b_ref[...]
    if relu:
        y = jnp.maximum(y, 0.0)
    return xp_ref[...] + y


def _nodemid_body(pm_ref, pd_ref, xp_ref, b_ref, p_ref, wl_ref, wr_ref,
                  xn_ref, xl_ref, xr_ref):
    xn = _combine(pm_ref, pd_ref, xp_ref, b_ref, p_ref, relu=True)
    xn_ref[...] = xn
    xl_ref[...] = jnp.dot(xn, wl_ref[...], preferred_element_type=jnp.float32)
    xr_ref[...] = jnp.dot(xn, wr_ref[...], preferred_element_type=jnp.float32)


def _final_body(pm_ref, pd_ref, xp_ref, b_ref, p_ref, out_ref):
    out_ref[...] = _combine(pm_ref, pd_ref, xp_ref, b_ref, p_ref, relu=False)


_W_SPEC = pl.BlockSpec((D, HF), lambda i: (0, 0))
_P_SPEC = pl.BlockSpec((H, HF), lambda i: (0, 0))
_B_SPEC = pl.BlockSpec((1, HF), lambda i: (0, 0))
_X_SPEC = pl.BlockSpec((RB, D), lambda i: (i, 0))
_PM_SPEC = pl.BlockSpec((1, RB, HF), lambda i: (i // 5, i % 5, 0))
_PD_SPEC = pl.BlockSpec((1, RB, H), lambda i: (i // 5, i % 5, 0))

_node0 = pl.pallas_call(
    _node0_body,
    grid=(GRID,),
    in_specs=[_X_SPEC, _W_SPEC, _W_SPEC],
    out_specs=[_X_SPEC, _X_SPEC],
    out_shape=[jax.ShapeDtypeStruct((N, HF), jnp.float32),
               jax.ShapeDtypeStruct((N, HF), jnp.float32)],
)

_nodemid = pl.pallas_call(
    _nodemid_body,
    grid=(GRID,),
    in_specs=[_PM_SPEC, _PD_SPEC, _X_SPEC, _B_SPEC, _P_SPEC, _W_SPEC, _W_SPEC],
    out_specs=[_X_SPEC, _X_SPEC, _X_SPEC],
    out_shape=[jax.ShapeDtypeStruct((N, HF), jnp.float32),
               jax.ShapeDtypeStruct((N, HF), jnp.float32),
               jax.ShapeDtypeStruct((N, HF), jnp.float32)],
)

_final = pl.pallas_call(
    _final_body,
    grid=(GRID,),
    in_specs=[_PM_SPEC, _PD_SPEC, _X_SPEC, _B_SPEC, _P_SPEC],
    out_specs=_X_SPEC,
    out_shape=jax.ShapeDtypeStruct((N, HF), jnp.float32),
)


@functools.partial(
    pl.kernel,
    out_type=(jax.ShapeDtypeStruct((NC, NRH, HF), jnp.float32),
              jax.ShapeDtypeStruct((NC, DR, 128), jnp.float32)),
    mesh=plsc.VectorSubcoreMesh(core_axis_name="c", subcore_axis_name="s"),
    compiler_params=pltpu.CompilerParams(needs_layout_passes=False),
    scratch_types=[
        pltpu.VMEM((3, K), jnp.int32),
        pltpu.VMEM((3, K), jnp.int32),
        pltpu.VMEM((K, HF), jnp.float32),
        pltpu.VMEM((K, HF), jnp.float32),
        pltpu.VMEM((K, HF), jnp.float32),
        pltpu.VMEM((K, HF), jnp.float32),
        pltpu.VMEM((K, HF), jnp.float32),
        pltpu.VMEM((K, HF), jnp.float32),
        pltpu.VMEM((K,), jnp.int32),
        pltpu.VMEM((K,), jnp.int32),
        pltpu.VMEM((H, F), jnp.float32),
        pltpu.VMEM((DR,), jnp.int32),
        pltpu.VMEM((16, F), jnp.float32),
        pltpu.VMEM((DR, 128), jnp.float32),
        pltpu.VMEM_SHARED((NRH, HF), jnp.float32),
        pltpu.VMEM_SHARED((DR, 128), jnp.float32),
        pltpu.SemaphoreType.DMA,
        pltpu.SemaphoreType.DMA,
        pltpu.SemaphoreType.DMA,
        pltpu.SemaphoreType.DMA,
        pltpu.SemaphoreType.DMA,
        pltpu.SemaphoreType.DMA,
        pltpu.SemaphoreType.DMA,
        pltpu.SemaphoreType.DMA,
    ],
)
def _edge_pass(edges_hbm, xl_hbm, xr_hbm, att_hbm, zeros_hbm, ridx_hbm,
               msg_hbm, den_hbm,
               idx0, idx1, xla0, xla1, xra0, xra1, ob0, ob1, dsl0, dsl1,
               attv, ridxv, pbuf, denl, accm, accd,
               si0, si1, sl0, sl1, sr0, sr1, ss0, ss1):
    c = lax.axis_index("c")
    s = lax.axis_index("s")
    r0 = s * RPT
    d0 = s * DPT
    pltpu.sync_copy(zeros_hbm.at[pl.ds(r0, RPT)], accm.at[pl.ds(r0, RPT)])
    pltpu.sync_copy(zeros_hbm.at[pl.ds(d0, DPT)], accd.at[pl.ds(d0, DPT)])
    pltpu.sync_copy(zeros_hbm.at[pl.ds(0, DR)], denl)
    pltpu.sync_copy(att_hbm, attv)
    pltpu.sync_copy(ridx_hbm, ridxv)
    plsc.subcore_barrier()

    IDX = [idx0, idx1]
    XLA = [xla0, xla1]
    XRA = [xra0, xra1]
    OB = [ob0, ob1]
    DSL = [dsl0, dsl1]
    SI = [si0, si1]
    SL = [sl0, sl1]
    SR = [sr0, sr1]
    SS = [ss0, ss1]

    gb0 = s * BLOCKS
    lane = lax.iota(jnp.int32, 16)
    lmask = lane < H
    nbase = c * HALF
    attrows = [attv[h, :] for h in range(H)]

    pltpu.sync_copy(edges_hbm.at[gb0], idx0)
    pltpu.async_copy(xl_hbm.at[idx0.at[0]], xla0, sl0)
    pltpu.async_copy(xr_hbm.at[idx0.at[1]], xra0, sr0)

    def pair(pb, carry):
        for i in range(2):
            o = i ^ 1
            bb = pb * 2 + i
            has_next = bb + 1 < BLOCKS

            @pl.when(has_next)
            def _():
                pltpu.async_copy(edges_hbm.at[gb0 + bb + 1], IDX[o], SI[o])

            pltpu.make_async_copy(xl_hbm.at[IDX[i].at[0]], XLA[i],
                                  SL[i]).wait()
            pltpu.make_async_copy(xr_hbm.at[IDX[i].at[1]], XRA[i],
                                  SR[i]).wait()

            @pl.when(bb >= 2)
            def _():
                pltpu.make_async_copy(OB[i], accm.at[DSL[i]], SS[i]).wait()

            dvv = IDX[i][2, :] - nbase
            keep = (dvv >= 0) & (dvv < HALF)
            DSL[i][...] = jnp.where(keep, dvv, JUNK)

            @pl.when(has_next)
            def _():
                pltpu.make_async_copy(edges_hbm.at[gb0 + bb + 1],
                                      IDX[o], SI[o]).wait()
                pltpu.async_copy(xl_hbm.at[IDX[o].at[0]], XLA[o], SL[o])
                pltpu.async_copy(xr_hbm.at[IDX[o].at[1]], XRA[o], SR[o])

            pvs = []
            for h in range(H):
                acc = jnp.zeros((16,), jnp.float32)
                ar = attrows[h]
                for f in range(F):
                    cfv = jnp.full((16,), h * F + f, jnp.int32)
                    xlv = plsc.load_gather(XLA[i], [lane, cfv])
                    xrv = plsc.load_gather(XRA[i], [lane, cfv])
                    u = xlv + xrv
                    lr = jnp.maximum(u, NEG_SLOPE * u)
                    acc = acc + ar[f] * lr
                pv = jnp.exp(acc)
                pvs.append(pv)
                pbuf[h, :] = pv

            for h in range(H):
                pv = pvs[h]
                for f in range(F):
                    cfv = jnp.full((16,), h * F + f, jnp.int32)
                    xlv = plsc.load_gather(XLA[i], [lane, cfv])
                    plsc.store_scatter(OB[i], [lane, cfv], pv * xlv)

            d16 = DSL[i][...]
            for j in range(16):
                dv = plsc.load_gather(pbuf, [lane, jnp.full((16,), j,
                                                            jnp.int32)])
                idx9 = d16[j] * H + lane
                plsc.addupdate_scatter(
                    denl, [idx9 >> 7, idx9 & 127], dv, mask=lmask)

            pltpu.async_copy(OB[i], accm.at[DSL[i]], SS[i], add=True)
        return carry

    lax.fori_loop(0, BLOCKS // 2, pair, 0)
    pltpu.make_async_copy(OB[0], accm.at[DSL[0]], SS[0]).wait()
    pltpu.make_async_copy(OB[1], accm.at[DSL[1]], SS[1]).wait()

    pltpu.sync_copy(denl, accd.at[ridxv], add=True)
    plsc.subcore_barrier()
    pltpu.sync_copy(accm.at[pl.ds(r0, RPT)], msg_hbm.at[c, pl.ds(r0, RPT)])
    pltpu.sync_copy(accd.at[pl.ds(d0, DPT)], den_hbm.at[c, pl.ds(d0, DPT)])


def kernel(z, edge_index, Wl0, Wr0, att0, b0, Wl1, Wr1, att1, b1,
           Wl2, Wr2, att2, b2, Wl3, Wr3, att3, b3):
    loop = jnp.arange(N, dtype=edge_index.dtype)
    src = jnp.concatenate([edge_index[0], loop])
    dst = jnp.concatenate([edge_index[1], loop])
    npad = ETP - ET
    src = jnp.concatenate([src, jnp.zeros((npad,), jnp.int32)])
    dstg = jnp.concatenate([dst, jnp.zeros((npad,), jnp.int32)])
    dsts = jnp.concatenate([dst, jnp.full((npad,), N, jnp.int32)])
    edges = jnp.stack([src, dstg, dsts])
    edges = edges.reshape(3, NBLK, K).transpose(1, 0, 2)

    sel = jnp.kron(jnp.eye(H, dtype=jnp.float32),
                   jnp.ones((1, F), jnp.float32))
    zeros_acc = jnp.zeros((NRH, HF), jnp.float32)
    ridx = jnp.arange(DR, dtype=jnp.int32)

    params = [(Wl0, Wr0, att0, b0), (Wl1, Wr1, att1, b1),
              (Wl2, Wr2, att2, b2), (Wl3, Wr3, att3, b3)]

    x = z
    pm = pd = None
    for i, (Wl, Wr, att, b) in enumerate(params):
        if i == 0:
            xl, xr = _node0(x, Wl, Wr)
        else:
            bprev = params[i - 1][3].reshape(1, HF)
            x, xl, xr = _nodemid(pm, pd, x, bprev, sel, Wl, Wr)
        msg, den = _edge_pass(edges, xl, xr, att, zeros_acc, ridx)
        pm = msg
        pd = den.reshape(NC, DR * 128 // H, H)

    blast = params[-1][3].reshape(1, HF)
    return _final(pm, pd, x, blast, sel)


# speedup vs baseline: 3.4135x; 3.4135x over previous
import functools

import jax
import jax.numpy as jnp
from jax import lax
from jax.experimental import pallas as pl
from jax.experimental.pallas import tpu as pltpu
from jax.experimental.pallas import tpu_sc as plsc

N = 10000
D = 128
H = 8
F = 16
HF = H * F
E = 320000
ET = E + N

NEG_SLOPE = 0.2

NC = 2
NS = 16
K = 16
BLOCKS = -(-ET // (NS * K))
BLOCKS += BLOCKS % 2
ETP = NS * K * BLOCKS
NBLK = NS * BLOCKS
HALF = 5000
NRH = 5120
RPT = NRH // NS
JUNK = NRH - 1
DR = 384
DPT = DR // NS

RB = 1000
GRID = N // RB


def _node0_body(x_ref, wl_ref, wr_ref, xl_ref, xr_ref):
    x = x_ref[...]
    xl_ref[...] = jnp.dot(x, wl_ref[...], preferred_element_type=jnp.float32)
    xr_ref[...] = jnp.dot(x, wr_ref[...], preferred_element_type=jnp.float32)


def _combine(pm_ref, pd_ref, xp_ref, b_ref, p_ref, relu):
    msg = pm_ref[0, :, :]
    den = pd_ref[0, :, :]
    recip = 1.0 / (den + 1e-16)
    rb = jnp.dot(recip, p_ref[...], preferred_element_type=jnp.float32)
    y = msg * rb + b_ref[...]
    if relu:
        y = jnp.maximum(y, 0.0)
    return xp_ref[...] + y


def _nodemid_body(pm_ref, pd_ref, xp_ref, b_ref, p_ref, wl_ref, wr_ref,
                  xn_ref, xl_ref, xr_ref):
    xn = _combine(pm_ref, pd_ref, xp_ref, b_ref, p_ref, relu=True)
    xn_ref[...] = xn
    xl_ref[...] = jnp.dot(xn, wl_ref[...], preferred_element_type=jnp.float32)
    xr_ref[...] = jnp.dot(xn, wr_ref[...], preferred_element_type=jnp.float32)


def _final_body(pm_ref, pd_ref, xp_ref, b_ref, p_ref, out_ref):
    out_ref[...] = _combine(pm_ref, pd_ref, xp_ref, b_ref, p_ref, relu=False)


_W_SPEC = pl.BlockSpec((D, HF), lambda i: (0, 0))
_P_SPEC = pl.BlockSpec((H, HF), lambda i: (0, 0))
_B_SPEC = pl.BlockSpec((1, HF), lambda i: (0, 0))
_X_SPEC = pl.BlockSpec((RB, D), lambda i: (i, 0))
_PM_SPEC = pl.BlockSpec((1, RB, HF), lambda i: (i // 5, i % 5, 0))
_PD_SPEC = pl.BlockSpec((1, RB, H), lambda i: (i // 5, i % 5, 0))

_node0 = pl.pallas_call(
    _node0_body,
    grid=(GRID,),
    in_specs=[_X_SPEC, _W_SPEC, _W_SPEC],
    out_specs=[_X_SPEC, _X_SPEC],
    out_shape=[jax.ShapeDtypeStruct((N, HF), jnp.float32),
               jax.ShapeDtypeStruct((N, HF), jnp.float32)],
)

_nodemid = pl.pallas_call(
    _nodemid_body,
    grid=(GRID,),
    in_specs=[_PM_SPEC, _PD_SPEC, _X_SPEC, _B_SPEC, _P_SPEC, _W_SPEC, _W_SPEC],
    out_specs=[_X_SPEC, _X_SPEC, _X_SPEC],
    out_shape=[jax.ShapeDtypeStruct((N, HF), jnp.float32),
               jax.ShapeDtypeStruct((N, HF), jnp.float32),
               jax.ShapeDtypeStruct((N, HF), jnp.float32)],
)

_final = pl.pallas_call(
    _final_body,
    grid=(GRID,),
    in_specs=[_PM_SPEC, _PD_SPEC, _X_SPEC, _B_SPEC, _P_SPEC],
    out_specs=_X_SPEC,
    out_shape=jax.ShapeDtypeStruct((N, HF), jnp.float32),
)


@functools.partial(
    pl.kernel,
    out_type=(jax.ShapeDtypeStruct((NC, NRH, HF), jnp.float32),
              jax.ShapeDtypeStruct((NC, DR, 128), jnp.float32)),
    mesh=plsc.VectorSubcoreMesh(core_axis_name="c", subcore_axis_name="s"),
    compiler_params=pltpu.CompilerParams(needs_layout_passes=False),
    scratch_types=[
        pltpu.VMEM((3, K), jnp.int32),
        pltpu.VMEM((3, K), jnp.int32),
        pltpu.VMEM((K, HF), jnp.float32),
        pltpu.VMEM((K, HF), jnp.float32),
        pltpu.VMEM((K, HF), jnp.float32),
        pltpu.VMEM((K, HF), jnp.float32),
        pltpu.VMEM((K, HF), jnp.float32),
        pltpu.VMEM((K, HF), jnp.float32),
        pltpu.VMEM((K,), jnp.int32),
        pltpu.VMEM((K,), jnp.int32),
        pltpu.VMEM((HF, F), jnp.float32),
        pltpu.VMEM((DR,), jnp.int32),
        pltpu.VMEM((16, F), jnp.float32),
        pltpu.VMEM((DR, 128), jnp.float32),
        pltpu.VMEM_SHARED((NRH, HF), jnp.float32),
        pltpu.VMEM_SHARED((DR, 128), jnp.float32),
        pltpu.SemaphoreType.DMA,
        pltpu.SemaphoreType.DMA,
        pltpu.SemaphoreType.DMA,
        pltpu.SemaphoreType.DMA,
        pltpu.SemaphoreType.DMA,
        pltpu.SemaphoreType.DMA,
        pltpu.SemaphoreType.DMA,
        pltpu.SemaphoreType.DMA,
    ],
)
def _edge_pass(edges_hbm, xl_hbm, xr_hbm, att_hbm, zeros_hbm, ridx_hbm,
               msg_hbm, den_hbm,
               idx0, idx1, xla0, xla1, xra0, xra1, ob0, ob1, dsl0, dsl1,
               attv, ridxv, pbuf, denl, accm, accd,
               si0, si1, sl0, sl1, sr0, sr1, ss0, ss1):
    c = lax.axis_index("c")
    s = lax.axis_index("s")
    r0 = s * RPT
    d0 = s * DPT
    pltpu.sync_copy(zeros_hbm.at[pl.ds(r0, RPT)], accm.at[pl.ds(r0, RPT)])
    pltpu.sync_copy(zeros_hbm.at[pl.ds(d0, DPT)], accd.at[pl.ds(d0, DPT)])
    pltpu.sync_copy(zeros_hbm.at[pl.ds(0, DR)], denl)
    pltpu.sync_copy(att_hbm, attv)
    pltpu.sync_copy(ridx_hbm, ridxv)
    plsc.subcore_barrier()

    IDX = [idx0, idx1]
    XLA = [xla0, xla1]
    XRA = [xra0, xra1]
    OB = [ob0, ob1]
    DSL = [dsl0, dsl1]
    SI = [si0, si1]
    SL = [sl0, sl1]
    SR = [sr0, sr1]
    SS = [ss0, ss1]

    gb0 = s * BLOCKS
    lane = lax.iota(jnp.int32, 16)
    lmask = lane < H
    nbase = c * HALF

    def colbase(f):
        return (lane + f) & 15

    pltpu.sync_copy(edges_hbm.at[gb0], idx0)
    pltpu.async_copy(xl_hbm.at[idx0.at[0]], xla0, sl0)
    pltpu.async_copy(xr_hbm.at[idx0.at[1]], xra0, sr0)

    def pair(pb, carry):
        for i in range(2):
            o = i ^ 1
            bb = pb * 2 + i
            has_next = bb + 1 < BLOCKS

            @pl.when(has_next)
            def _():
                pltpu.async_copy(edges_hbm.at[gb0 + bb + 1], IDX[o], SI[o])

            pltpu.make_async_copy(xl_hbm.at[IDX[i].at[0]], XLA[i],
                                  SL[i]).wait()
            pltpu.make_async_copy(xr_hbm.at[IDX[i].at[1]], XRA[i],
                                  SR[i]).wait()

            @pl.when(bb >= 2)
            def _():
                pltpu.make_async_copy(OB[i], accm.at[DSL[i]], SS[i]).wait()

            dvv = IDX[i][2, :] - nbase
            keep = (dvv >= 0) & (dvv < HALF)
            DSL[i][...] = jnp.where(keep, dvv, JUNK)

            @pl.when(has_next)
            def _():
                pltpu.make_async_copy(edges_hbm.at[gb0 + bb + 1],
                                      IDX[o], SI[o]).wait()
                pltpu.async_copy(xl_hbm.at[IDX[o].at[0]], XLA[o], SL[o])
                pltpu.async_copy(xr_hbm.at[IDX[o].at[1]], XRA[o], SR[o])

            xla_i = XLA[i]
            xra_i = XRA[i]
            ob_i = OB[i]
            dsl_i = DSL[i]

            def hloop(h, hc):
                acc = jnp.zeros((16,), jnp.float32)
                hb = h * F
                for f in range(F):
                    cfv = colbase(f) + hb
                    xlv = plsc.load_gather(xla_i, [lane, cfv])
                    xrv = plsc.load_gather(xra_i, [lane, cfv])
                    u = xlv + xrv
                    lr = jnp.maximum(u, NEG_SLOPE * u)
                    acc = acc + attv[hb + f, :] * lr
                pv = jnp.exp(acc)
                plsc.store_scatter(pbuf, [jnp.full((16,), 0, jnp.int32) + h,
                                          colbase(h)], pv)
                return hc

            lax.fori_loop(0, H, hloop, 0)

            def jloop(j, jc):
                dv = plsc.load_gather(pbuf, [lane, colbase(j)])
                for h in range(H):
                    ph = jnp.full((16,), dv[h])
                    ob_i[j, pl.ds(h * F, 16)] = (
                        ph * xla_i[j, pl.ds(h * F, 16)])
                dstv = plsc.load_gather(dsl_i,
                                        [jnp.full((16,), 0, jnp.int32) + j])
                idx9 = dstv * H + lane
                plsc.addupdate_scatter(
                    denl, [idx9 >> 7, idx9 & 127], dv, mask=lmask)
                return jc

            lax.fori_loop(0, 16, jloop, 0)

            pltpu.async_copy(OB[i], accm.at[DSL[i]], SS[i], add=True)
        return carry

    lax.fori_loop(0, BLOCKS // 2, pair, 0)
    pltpu.make_async_copy(OB[0], accm.at[DSL[0]], SS[0]).wait()
    pltpu.make_async_copy(OB[1], accm.at[DSL[1]], SS[1]).wait()

    pltpu.sync_copy(denl, accd.at[ridxv], add=True)
    plsc.subcore_barrier()
    pltpu.sync_copy(accm.at[pl.ds(r0, RPT)], msg_hbm.at[c, pl.ds(r0, RPT)])
    pltpu.sync_copy(accd.at[pl.ds(d0, DPT)], den_hbm.at[c, pl.ds(d0, DPT)])


def kernel(z, edge_index, Wl0, Wr0, att0, b0, Wl1, Wr1, att1, b1,
           Wl2, Wr2, att2, b2, Wl3, Wr3, att3, b3):
    loop = jnp.arange(N, dtype=edge_index.dtype)
    src = jnp.concatenate([edge_index[0], loop])
    dst = jnp.concatenate([edge_index[1], loop])
    npad = ETP - ET
    src = jnp.concatenate([src, jnp.zeros((npad,), jnp.int32)])
    dstg = jnp.concatenate([dst, jnp.zeros((npad,), jnp.int32)])
    dsts = jnp.concatenate([dst, jnp.full((npad,), N, jnp.int32)])
    edges = jnp.stack([src, dstg, dsts])
    edges = edges.reshape(3, NBLK, K).transpose(1, 0, 2)

    sel = jnp.kron(jnp.eye(H, dtype=jnp.float32),
                   jnp.ones((1, F), jnp.float32))
    fl = (jnp.arange(F)[:, None] + jnp.arange(16)[None, :]) % F
    zeros_acc = jnp.zeros((NRH, HF), jnp.float32)
    ridx = jnp.arange(DR, dtype=jnp.int32)

    params = [(Wl0, Wr0, att0, b0), (Wl1, Wr1, att1, b1),
              (Wl2, Wr2, att2, b2), (Wl3, Wr3, att3, b3)]

    x = z
    pm = pd = None
    for i, (Wl, Wr, att, b) in enumerate(params):
        if i == 0:
            xl, xr = _node0(x, Wl, Wr)
        else:
            bprev = params[i - 1][3].reshape(1, HF)
            x, xl, xr = _nodemid(pm, pd, x, bprev, sel, Wl, Wr)
        att_sk = att[:, fl].reshape(HF, 16)
        msg, den = _edge_pass(edges, xl, xr, att_sk, zeros_acc, ridx)
        pm = msg
        pd = den.reshape(NC, DR * 128 // H, H)

    blast = params[-1][3].reshape(1, HF)
    return _final(pm, pd, x, blast, sel)
